# initial kernel scaffold (unmeasured)
import jax
import jax.numpy as jnp
from jax import lax
from jax.experimental import pallas as pl
from jax.experimental.pallas import tpu as pltpu

N_DEV = 8
M_TILE = 512


def kernel(x, w_mat, scale_x, scale_w):
    m, k_per = x.shape
    _, n = w_mat.shape
    n_tiles = m // M_TILE

    def body(x_ref, w_ref, sx_ref, sw_ref, out_ref,
             cx_ref, cw_ref, sx_send, sx_recv, sw_send, sw_recv, credit_sem):
        my = lax.axis_index("i")
        left = lax.rem(my + N_DEV - 1, N_DEV)
        right = lax.rem(my + 1, N_DEV)

        barrier_sem = pltpu.get_barrier_semaphore()
        for nbr in (left, right):
            pl.semaphore_signal(
                barrier_sem, inc=1,
                device_id=(nbr,), device_id_type=pl.DeviceIdType.MESH,
            )
        pl.semaphore_wait(barrier_sem, 2)

        def accum_gemm(xc, wc, first):
            def tile_body(i, _):
                sl = pl.ds(i * M_TILE, M_TILE)
                part = jnp.dot(xc[sl, :], wc[:, :],
                               preferred_element_type=jnp.float32)
                if first:
                    out_ref[sl, :] = part
                else:
                    out_ref[sl, :] = out_ref[sl, :] + part
                return 0
            lax.fori_loop(0, n_tiles, tile_body, 0)

        accum_gemm(x_ref, w_ref, first=True)

        for h in range(N_DEV - 1):
            r_slot = h % 3
            if h >= 3:
                pl.semaphore_wait(credit_sem, 1)
            if h == 0:
                src_x, src_w = x_ref, w_ref
            else:
                src_x = cx_ref.at[(h - 1) % 3]
                src_w = cw_ref.at[(h - 1) % 3]
            rdma_x = pltpu.make_async_remote_copy(
                src_ref=src_x, dst_ref=cx_ref.at[r_slot],
                send_sem=sx_send.at[h], recv_sem=sx_recv.at[h],
                device_id=(right,), device_id_type=pl.DeviceIdType.MESH,
            )
            rdma_w = pltpu.make_async_remote_copy(
                src_ref=src_w, dst_ref=cw_ref.at[r_slot],
                send_sem=sw_send.at[h], recv_sem=sw_recv.at[h],
                device_id=(right,), device_id_type=pl.DeviceIdType.MESH,
            )
            rdma_x.start()
            rdma_w.start()
            rdma_x.wait()
            rdma_w.wait()
            if 1 <= h <= 4:
                pl.semaphore_signal(
                    credit_sem, inc=1,
                    device_id=(left,), device_id_type=pl.DeviceIdType.MESH,
                )
            accum_gemm(cx_ref.at[r_slot], cw_ref.at[r_slot], first=False)

        s = sx_ref[0] * sw_ref[0]

        def epi_body(i, _):
            sl = pl.ds(i * M_TILE, M_TILE)
            y = out_ref[sl, :] * s
            out_ref[sl, :] = y * jax.nn.sigmoid(y)
            return 0
        lax.fori_loop(0, n_tiles, epi_body, 0)

    return pl.pallas_call(
        body,
        out_shape=jax.ShapeDtypeStruct((m, n), jnp.float32),
        in_specs=[
            pl.BlockSpec(memory_space=pltpu.VMEM),
            pl.BlockSpec(memory_space=pltpu.VMEM),
            pl.BlockSpec(memory_space=pltpu.SMEM),
            pl.BlockSpec(memory_space=pltpu.SMEM),
        ],
        out_specs=pl.BlockSpec(memory_space=pltpu.VMEM),
        scratch_shapes=[
            pltpu.VMEM((3, m, k_per), x.dtype),
            pltpu.VMEM((3, k_per, n), w_mat.dtype),
            pltpu.SemaphoreType.DMA((N_DEV - 1,)),
            pltpu.SemaphoreType.DMA((N_DEV - 1,)),
            pltpu.SemaphoreType.DMA((N_DEV - 1,)),
            pltpu.SemaphoreType.DMA((N_DEV - 1,)),
            pltpu.SemaphoreType.REGULAR,
        ],
        compiler_params=pltpu.CompilerParams(collective_id=0),
    )(x, w_mat, scale_x, scale_w)


# baseline (device time: 358405 ns/iter reference)
import jax
import jax.numpy as jnp
from jax import lax
from jax.experimental import pallas as pl
from jax.experimental.pallas import tpu as pltpu

N_DEV = 8
M_TILE = 512


def kernel(x, w_mat, scale_x, scale_w):
    x = x.astype(jnp.float8_e5m2)
    w_mat = w_mat.astype(jnp.float8_e5m2)
    m, k_per = x.shape
    _, n = w_mat.shape
    n_tiles = m // M_TILE

    def body(x_ref, w_ref, sx_ref, sw_ref, out_ref,
             cx_ref, cw_ref, sx_send, sx_recv, sw_send, sw_recv, credit_sem):
        my = lax.axis_index("i")
        left = lax.rem(my + N_DEV - 1, N_DEV)
        right = lax.rem(my + 1, N_DEV)

        barrier_sem = pltpu.get_barrier_semaphore()
        for nbr in (left, right):
            pl.semaphore_signal(
                barrier_sem, inc=1,
                device_id=(nbr,), device_id_type=pl.DeviceIdType.MESH,
            )
        pl.semaphore_wait(barrier_sem, 2)

        def accum_gemm(xc, wc, first):
            def tile_body(i, _):
                sl = pl.ds(i * M_TILE, M_TILE)
                part = jnp.dot(xc[sl, :], wc[:, :],
                               preferred_element_type=jnp.float32)
                if first:
                    out_ref[sl, :] = part
                else:
                    out_ref[sl, :] = out_ref[sl, :] + part
                return 0
            lax.fori_loop(0, n_tiles, tile_body, 0)

        accum_gemm(x_ref, w_ref, first=True)

        for h in range(N_DEV - 1):
            r_slot = h % 3
            if h >= 3:
                pl.semaphore_wait(credit_sem, 1)
            if h == 0:
                src_x, src_w = x_ref, w_ref
            else:
                src_x = cx_ref.at[(h - 1) % 3]
                src_w = cw_ref.at[(h - 1) % 3]
            rdma_x = pltpu.make_async_remote_copy(
                src_ref=src_x, dst_ref=cx_ref.at[r_slot],
                send_sem=sx_send.at[h], recv_sem=sx_recv.at[h],
                device_id=(right,), device_id_type=pl.DeviceIdType.MESH,
            )
            rdma_w = pltpu.make_async_remote_copy(
                src_ref=src_w, dst_ref=cw_ref.at[r_slot],
                send_sem=sw_send.at[h], recv_sem=sw_recv.at[h],
                device_id=(right,), device_id_type=pl.DeviceIdType.MESH,
            )
            rdma_x.start()
            rdma_w.start()
            rdma_x.wait()
            rdma_w.wait()
            if 1 <= h <= 4:
                pl.semaphore_signal(
                    credit_sem, inc=1,
                    device_id=(left,), device_id_type=pl.DeviceIdType.MESH,
                )
            accum_gemm(cx_ref.at[r_slot], cw_ref.at[r_slot], first=False)

        s = sx_ref[0] * sw_ref[0]

        def epi_body(i, _):
            sl = pl.ds(i * M_TILE, M_TILE)
            y = out_ref[sl, :] * s
            out_ref[sl, :] = y * jax.nn.sigmoid(y)
            return 0
        lax.fori_loop(0, n_tiles, epi_body, 0)

    return pl.pallas_call(
        body,
        out_shape=jax.ShapeDtypeStruct((m, n), jnp.float32),
        in_specs=[
            pl.BlockSpec(memory_space=pltpu.VMEM),
            pl.BlockSpec(memory_space=pltpu.VMEM),
            pl.BlockSpec(memory_space=pltpu.SMEM),
            pl.BlockSpec(memory_space=pltpu.SMEM),
        ],
        out_specs=pl.BlockSpec(memory_space=pltpu.VMEM),
        scratch_shapes=[
            pltpu.VMEM((3, m, k_per), x.dtype),
            pltpu.VMEM((3, k_per, n), w_mat.dtype),
            pltpu.SemaphoreType.DMA((N_DEV - 1,)),
            pltpu.SemaphoreType.DMA((N_DEV - 1,)),
            pltpu.SemaphoreType.DMA((N_DEV - 1,)),
            pltpu.SemaphoreType.DMA((N_DEV - 1,)),
            pltpu.SemaphoreType.REGULAR,
        ],
        compiler_params=pltpu.CompilerParams(
            collective_id=0,
            vmem_limit_bytes=60 * 1024 * 1024,
        ),
    )(x, w_mat, scale_x, scale_w)


# device time: 202212 ns/iter; 1.7724x vs baseline; 1.7724x over previous
import jax
import jax.numpy as jnp
from jax import lax
from jax.experimental import pallas as pl
from jax.experimental.pallas import tpu as pltpu

N_DEV = 8
N_HOP = N_DEV - 1
M_TILE = 512
N_SLOT = 3


def kernel(x, w_mat, scale_x, scale_w):
    x = x.astype(jnp.float8_e5m2)
    w_mat = w_mat.astype(jnp.float8_e5m2)
    m, k_per = x.shape
    _, n = w_mat.shape
    kh = k_per // 2
    x_a, x_b = x[:, :kh], x[:, kh:]
    w_a, w_b = w_mat[:kh, :], w_mat[kh:, :]
    n_tiles = m // M_TILE

    def body(xa_ref, xb_ref, wa_ref, wb_ref, sx_ref, sw_ref, out_ref,
             cxr_ref, cwr_ref, cxl_ref, cwl_ref,
             rxs, rxr, rws, rwr, lxs, lxr, lws, lwr,
             credit_cw, credit_ccw):
        my = lax.axis_index("i")
        left = lax.rem(my + N_DEV - 1, N_DEV)
        right = lax.rem(my + 1, N_DEV)

        barrier_sem = pltpu.get_barrier_semaphore()
        for nbr in (left, right):
            pl.semaphore_signal(
                barrier_sem, inc=1,
                device_id=(nbr,), device_id_type=pl.DeviceIdType.MESH,
            )
        pl.semaphore_wait(barrier_sem, 2)

        def accum_gemm(xc, wc, first):
            def tile_body(i, _):
                sl = pl.ds(i * M_TILE, M_TILE)
                part = jnp.dot(xc[sl, :], wc[:, :],
                               preferred_element_type=jnp.float32)
                if first:
                    out_ref[sl, :] = part
                else:
                    out_ref[sl, :] = out_ref[sl, :] + part
                return 0
            lax.fori_loop(0, n_tiles, tile_body, 0)

        def rdma(src, dst, ssem, rsem, target):
            return pltpu.make_async_remote_copy(
                src_ref=src, dst_ref=dst, send_sem=ssem, recv_sem=rsem,
                device_id=(target,), device_id_type=pl.DeviceIdType.MESH,
            )

        for h in range(N_HOP):
            r_slot = h % N_SLOT
            if h >= N_SLOT:
                pl.semaphore_wait(credit_cw, 1)
                pl.semaphore_wait(credit_ccw, 1)
            if h == 0:
                sxa, swa = xa_ref, wa_ref
                sxb, swb = xb_ref, wb_ref
            else:
                p = (h - 1) % N_SLOT
                sxa, swa = cxr_ref.at[p], cwr_ref.at[p]
                sxb, swb = cxl_ref.at[p], cwl_ref.at[p]
            cp = [
                rdma(sxa, cxr_ref.at[r_slot], rxs.at[r_slot], rxr.at[r_slot], right),
                rdma(swa, cwr_ref.at[r_slot], rws.at[r_slot], rwr.at[r_slot], right),
                rdma(sxb, cxl_ref.at[r_slot], lxs.at[r_slot], lxr.at[r_slot], left),
                rdma(swb, cwl_ref.at[r_slot], lws.at[r_slot], lwr.at[r_slot], left),
            ]
            for c in cp:
                c.start()
            if h == 0:
                accum_gemm(xa_ref, wa_ref, first=True)
                accum_gemm(xb_ref, wb_ref, first=False)
            else:
                p = (h - 1) % N_SLOT
                accum_gemm(cxr_ref.at[p], cwr_ref.at[p], first=False)
                accum_gemm(cxl_ref.at[p], cwl_ref.at[p], first=False)
            for c in cp:
                c.wait_recv()
            for c in cp:
                c.wait_send()
            if 1 <= h <= 4:
                pl.semaphore_signal(
                    credit_cw, inc=1,
                    device_id=(left,), device_id_type=pl.DeviceIdType.MESH,
                )
                pl.semaphore_signal(
                    credit_ccw, inc=1,
                    device_id=(right,), device_id_type=pl.DeviceIdType.MESH,
                )

        p = (N_HOP - 1) % N_SLOT
        accum_gemm(cxr_ref.at[p], cwr_ref.at[p], first=False)
        accum_gemm(cxl_ref.at[p], cwl_ref.at[p], first=False)

        s = sx_ref[0] * sw_ref[0]

        def epi_body(i, _):
            sl = pl.ds(i * M_TILE, M_TILE)
            y = out_ref[sl, :] * s
            out_ref[sl, :] = y * jax.nn.sigmoid(y)
            return 0
        lax.fori_loop(0, n_tiles, epi_body, 0)

    return pl.pallas_call(
        body,
        out_shape=jax.ShapeDtypeStruct((m, n), jnp.float32),
        in_specs=[
            pl.BlockSpec(memory_space=pltpu.VMEM),
            pl.BlockSpec(memory_space=pltpu.VMEM),
            pl.BlockSpec(memory_space=pltpu.VMEM),
            pl.BlockSpec(memory_space=pltpu.VMEM),
            pl.BlockSpec(memory_space=pltpu.SMEM),
            pl.BlockSpec(memory_space=pltpu.SMEM),
        ],
        out_specs=pl.BlockSpec(memory_space=pltpu.VMEM),
        scratch_shapes=[
            pltpu.VMEM((N_SLOT, m, kh), x.dtype),
            pltpu.VMEM((N_SLOT, kh, n), w_mat.dtype),
            pltpu.VMEM((N_SLOT, m, kh), x.dtype),
            pltpu.VMEM((N_SLOT, kh, n), w_mat.dtype),
            pltpu.SemaphoreType.DMA((N_SLOT,)),
            pltpu.SemaphoreType.DMA((N_SLOT,)),
            pltpu.SemaphoreType.DMA((N_SLOT,)),
            pltpu.SemaphoreType.DMA((N_SLOT,)),
            pltpu.SemaphoreType.DMA((N_SLOT,)),
            pltpu.SemaphoreType.DMA((N_SLOT,)),
            pltpu.SemaphoreType.DMA((N_SLOT,)),
            pltpu.SemaphoreType.DMA((N_SLOT,)),
            pltpu.SemaphoreType.REGULAR,
            pltpu.SemaphoreType.REGULAR,
        ],
        compiler_params=pltpu.CompilerParams(
            collective_id=0,
            vmem_limit_bytes=60 * 1024 * 1024,
        ),
    )(x_a, x_b, w_a, w_b, scale_x, scale_w)


# device time: 185996 ns/iter; 1.9270x vs baseline; 1.0872x over previous
import jax
import jax.numpy as jnp
from jax import lax
from jax.experimental import pallas as pl
from jax.experimental.pallas import tpu as pltpu

N_DEV = 8
N_HOP = N_DEV - 1
M_TILE = 512
N_SLOT = 3


def kernel(x, w_mat, scale_x, scale_w):
    x = x.astype(jnp.float8_e5m2)
    w_mat = w_mat.astype(jnp.float8_e5m2)
    m, k_per = x.shape
    _, n = w_mat.shape
    kh = k_per // 2
    n_tiles = m // M_TILE

    def body(x_ref, w_ref, sx_ref, sw_ref, out_hbm,
             acc_ref, oxa, oxb, owa, owb,
             cxr_ref, cwr_ref, cxl_ref, cwl_ref,
             rxs, rxr, rws, rwr, lxs, lxr, lws, lwr,
             out_sems, credit_cw, credit_ccw):
        my = lax.axis_index("i")
        left = lax.rem(my + N_DEV - 1, N_DEV)
        right = lax.rem(my + 1, N_DEV)

        barrier_sem = pltpu.get_barrier_semaphore()
        for nbr in (left, right):
            pl.semaphore_signal(
                barrier_sem, inc=1,
                device_id=(nbr,), device_id_type=pl.DeviceIdType.MESH,
            )
        pl.semaphore_wait(barrier_sem, 2)

        oxa[...] = x_ref[:, :kh]
        oxb[...] = x_ref[:, kh:]
        owa[...] = w_ref[:kh, :]
        owb[...] = w_ref[kh:, :]

        def accum_gemm(xc, wc, first):
            def tile_body(i, _):
                sl = pl.ds(i * M_TILE, M_TILE)
                part = jnp.dot(xc[sl, :], wc[:, :],
                               preferred_element_type=jnp.float32)
                if first:
                    acc_ref[sl, :] = part
                else:
                    acc_ref[sl, :] = acc_ref[sl, :] + part
                return 0
            lax.fori_loop(0, n_tiles, tile_body, 0)

        def rdma(src, dst, ssem, rsem, target):
            return pltpu.make_async_remote_copy(
                src_ref=src, dst_ref=dst, send_sem=ssem, recv_sem=rsem,
                device_id=(target,), device_id_type=pl.DeviceIdType.MESH,
            )

        for h in range(N_HOP):
            r_slot = h % N_SLOT
            if h >= N_SLOT:
                pl.semaphore_wait(credit_cw, 1)
                pl.semaphore_wait(credit_ccw, 1)
            if h == 0:
                sxa, swa, sxb, swb = oxa, owa, oxb, owb
            else:
                p = (h - 1) % N_SLOT
                sxa, swa = cxr_ref.at[p], cwr_ref.at[p]
                sxb, swb = cxl_ref.at[p], cwl_ref.at[p]
            cp = [
                rdma(sxa, cxr_ref.at[r_slot], rxs.at[r_slot], rxr.at[r_slot], right),
                rdma(swa, cwr_ref.at[r_slot], rws.at[r_slot], rwr.at[r_slot], right),
                rdma(sxb, cxl_ref.at[r_slot], lxs.at[r_slot], lxr.at[r_slot], left),
                rdma(swb, cwl_ref.at[r_slot], lws.at[r_slot], lwr.at[r_slot], left),
            ]
            for c in cp:
                c.start()
            if h == 0:
                accum_gemm(oxa, owa, first=True)
                accum_gemm(oxb, owb, first=False)
            else:
                p = (h - 1) % N_SLOT
                accum_gemm(cxr_ref.at[p], cwr_ref.at[p], first=False)
                accum_gemm(cxl_ref.at[p], cwl_ref.at[p], first=False)
            for c in cp:
                c.wait_recv()
            for c in cp:
                c.wait_send()
            if 1 <= h <= 4:
                pl.semaphore_signal(
                    credit_cw, inc=1,
                    device_id=(left,), device_id_type=pl.DeviceIdType.MESH,
                )
                pl.semaphore_signal(
                    credit_ccw, inc=1,
                    device_id=(right,), device_id_type=pl.DeviceIdType.MESH,
                )

        p = (N_HOP - 1) % N_SLOT
        s = sx_ref[0] * sw_ref[0]
        out_copies = []
        for i in range(n_tiles):
            sl = pl.ds(i * M_TILE, M_TILE)
            acc = (acc_ref[sl, :]
                   + jnp.dot(cxr_ref[p, sl, :], cwr_ref[p, :, :],
                             preferred_element_type=jnp.float32)
                   + jnp.dot(cxl_ref[p, sl, :], cwl_ref[p, :, :],
                             preferred_element_type=jnp.float32))
            y = acc * s
            acc_ref[sl, :] = y * jax.nn.sigmoid(y)
            cp = pltpu.make_async_copy(
                acc_ref.at[sl, :], out_hbm.at[sl, :], out_sems.at[i])
            cp.start()
            out_copies.append(cp)
        for cp in out_copies:
            cp.wait()

    return pl.pallas_call(
        body,
        out_shape=jax.ShapeDtypeStruct((m, n), jnp.float32),
        in_specs=[
            pl.BlockSpec(memory_space=pltpu.VMEM),
            pl.BlockSpec(memory_space=pltpu.VMEM),
            pl.BlockSpec(memory_space=pltpu.SMEM),
            pl.BlockSpec(memory_space=pltpu.SMEM),
        ],
        out_specs=pl.BlockSpec(memory_space=pl.ANY),
        scratch_shapes=[
            pltpu.VMEM((m, n), jnp.float32),
            pltpu.VMEM((m, kh), x.dtype),
            pltpu.VMEM((m, kh), x.dtype),
            pltpu.VMEM((kh, n), w_mat.dtype),
            pltpu.VMEM((kh, n), w_mat.dtype),
            pltpu.VMEM((N_SLOT, m, kh), x.dtype),
            pltpu.VMEM((N_SLOT, kh, n), w_mat.dtype),
            pltpu.VMEM((N_SLOT, m, kh), x.dtype),
            pltpu.VMEM((N_SLOT, kh, n), w_mat.dtype),
            pltpu.SemaphoreType.DMA((N_SLOT,)),
            pltpu.SemaphoreType.DMA((N_SLOT,)),
            pltpu.SemaphoreType.DMA((N_SLOT,)),
            pltpu.SemaphoreType.DMA((N_SLOT,)),
            pltpu.SemaphoreType.DMA((N_SLOT,)),
            pltpu.SemaphoreType.DMA((N_SLOT,)),
            pltpu.SemaphoreType.DMA((N_SLOT,)),
            pltpu.SemaphoreType.DMA((N_SLOT,)),
            pltpu.SemaphoreType.DMA((8,)),
            pltpu.SemaphoreType.REGULAR,
            pltpu.SemaphoreType.REGULAR,
        ],
        compiler_params=pltpu.CompilerParams(
            collective_id=0,
            vmem_limit_bytes=60 * 1024 * 1024,
        ),
    )(x, w_mat, scale_x, scale_w)


# device time: 180097 ns/iter; 1.9901x vs baseline; 1.0328x over previous
import jax
import jax.numpy as jnp
from jax import lax
from jax.experimental import pallas as pl
from jax.experimental.pallas import tpu as pltpu

N_DEV = 8
N_HOP = N_DEV - 1
M_TILE = 512
N_SLOT = 3


def kernel(x, w_mat, scale_x, scale_w):
    m, k_per = x.shape
    _, n = w_mat.shape
    kh = k_per // 2
    n_tiles = m // M_TILE

    def body(x_ref, w_ref, sx_ref, sw_ref, out_hbm,
             acc_ref, oxa, oxb, owa, owb,
             cxr_ref, cwr_ref, cxl_ref, cwl_ref,
             rxs, rxr, rws, rwr, lxs, lxr, lws, lwr,
             out_sems, credit_cw, credit_ccw):
        my = lax.axis_index("i")
        left = lax.rem(my + N_DEV - 1, N_DEV)
        right = lax.rem(my + 1, N_DEV)

        barrier_sem = pltpu.get_barrier_semaphore()
        for nbr in (left, right):
            pl.semaphore_signal(
                barrier_sem, inc=1,
                device_id=(nbr,), device_id_type=pl.DeviceIdType.MESH,
            )
        pl.semaphore_wait(barrier_sem, 2)

        oxa[...] = x_ref[:, :kh].astype(jnp.float8_e5m2)
        oxb[...] = x_ref[:, kh:].astype(jnp.float8_e5m2)
        owa[...] = w_ref[:kh, :].astype(jnp.float8_e5m2)
        owb[...] = w_ref[kh:, :].astype(jnp.float8_e5m2)

        def accum_gemm(xc, wc, first):
            def tile_body(i, _):
                sl = pl.ds(i * M_TILE, M_TILE)
                part = jnp.dot(xc[sl, :], wc[:, :],
                               preferred_element_type=jnp.float32)
                if first:
                    acc_ref[sl, :] = part
                else:
                    acc_ref[sl, :] = acc_ref[sl, :] + part
                return 0
            lax.fori_loop(0, n_tiles, tile_body, 0)

        def rdma(src, dst, ssem, rsem, target):
            return pltpu.make_async_remote_copy(
                src_ref=src, dst_ref=dst, send_sem=ssem, recv_sem=rsem,
                device_id=(target,), device_id_type=pl.DeviceIdType.MESH,
            )

        for h in range(N_HOP):
            r_slot = h % N_SLOT
            if h >= N_SLOT:
                pl.semaphore_wait(credit_cw, 1)
                pl.semaphore_wait(credit_ccw, 1)
            if h == 0:
                sxa, swa, sxb, swb = oxa, owa, oxb, owb
            else:
                p = (h - 1) % N_SLOT
                sxa, swa = cxr_ref.at[p], cwr_ref.at[p]
                sxb, swb = cxl_ref.at[p], cwl_ref.at[p]
            cp = [
                rdma(sxa, cxr_ref.at[r_slot], rxs.at[r_slot], rxr.at[r_slot], right),
                rdma(swa, cwr_ref.at[r_slot], rws.at[r_slot], rwr.at[r_slot], right),
                rdma(sxb, cxl_ref.at[r_slot], lxs.at[r_slot], lxr.at[r_slot], left),
                rdma(swb, cwl_ref.at[r_slot], lws.at[r_slot], lwr.at[r_slot], left),
            ]
            for c in cp:
                c.start()
            if h == 0:
                accum_gemm(oxa, owa, first=True)
                accum_gemm(oxb, owb, first=False)
            else:
                p = (h - 1) % N_SLOT
                accum_gemm(cxr_ref.at[p], cwr_ref.at[p], first=False)
                accum_gemm(cxl_ref.at[p], cwl_ref.at[p], first=False)
            for c in cp:
                c.wait_recv()
            for c in cp:
                c.wait_send()
            if 1 <= h <= 4:
                pl.semaphore_signal(
                    credit_cw, inc=1,
                    device_id=(left,), device_id_type=pl.DeviceIdType.MESH,
                )
                pl.semaphore_signal(
                    credit_ccw, inc=1,
                    device_id=(right,), device_id_type=pl.DeviceIdType.MESH,
                )

        p = (N_HOP - 1) % N_SLOT
        s = sx_ref[0] * sw_ref[0]
        out_copies = []
        for i in range(n_tiles):
            sl = pl.ds(i * M_TILE, M_TILE)
            acc = (acc_ref[sl, :]
                   + jnp.dot(cxr_ref[p, sl, :], cwr_ref[p, :, :],
                             preferred_element_type=jnp.float32)
                   + jnp.dot(cxl_ref[p, sl, :], cwl_ref[p, :, :],
                             preferred_element_type=jnp.float32))
            y = acc * s
            acc_ref[sl, :] = y * jax.nn.sigmoid(y)
            cp = pltpu.make_async_copy(
                acc_ref.at[sl, :], out_hbm.at[sl, :], out_sems.at[i])
            cp.start()
            out_copies.append(cp)
        for cp in out_copies:
            cp.wait()

    return pl.pallas_call(
        body,
        out_shape=jax.ShapeDtypeStruct((m, n), jnp.float32),
        in_specs=[
            pl.BlockSpec(memory_space=pltpu.VMEM),
            pl.BlockSpec(memory_space=pltpu.VMEM),
            pl.BlockSpec(memory_space=pltpu.SMEM),
            pl.BlockSpec(memory_space=pltpu.SMEM),
        ],
        out_specs=pl.BlockSpec(memory_space=pl.ANY),
        scratch_shapes=[
            pltpu.VMEM((m, n), jnp.float32),
            pltpu.VMEM((m, kh), jnp.float8_e5m2),
            pltpu.VMEM((m, kh), jnp.float8_e5m2),
            pltpu.VMEM((kh, n), jnp.float8_e5m2),
            pltpu.VMEM((kh, n), jnp.float8_e5m2),
            pltpu.VMEM((N_SLOT, m, kh), jnp.float8_e5m2),
            pltpu.VMEM((N_SLOT, kh, n), jnp.float8_e5m2),
            pltpu.VMEM((N_SLOT, m, kh), jnp.float8_e5m2),
            pltpu.VMEM((N_SLOT, kh, n), jnp.float8_e5m2),
            pltpu.SemaphoreType.DMA((N_SLOT,)),
            pltpu.SemaphoreType.DMA((N_SLOT,)),
            pltpu.SemaphoreType.DMA((N_SLOT,)),
            pltpu.SemaphoreType.DMA((N_SLOT,)),
            pltpu.SemaphoreType.DMA((N_SLOT,)),
            pltpu.SemaphoreType.DMA((N_SLOT,)),
            pltpu.SemaphoreType.DMA((N_SLOT,)),
            pltpu.SemaphoreType.DMA((N_SLOT,)),
            pltpu.SemaphoreType.DMA((8,)),
            pltpu.SemaphoreType.REGULAR,
            pltpu.SemaphoreType.REGULAR,
        ],
        compiler_params=pltpu.CompilerParams(
            collective_id=0,
            vmem_limit_bytes=62 * 1024 * 1024,
        ),
    )(x, w_mat, scale_x, scale_w)


# device time: 171536 ns/iter; 2.0894x vs baseline; 1.0499x over previous
import jax
import jax.numpy as jnp
from jax import lax
from jax.experimental import pallas as pl
from jax.experimental.pallas import tpu as pltpu

N_DEV = 8
N_HOP = N_DEV - 1
M_TILE = 512
N_SLOT = 3
MH = 2048


def kernel(x, w_mat, scale_x, scale_w):
    m, k_per = x.shape
    _, n = w_mat.shape
    kh = k_per // 2
    n_tiles = m // M_TILE

    def body(x_ref, w_ref, sx_ref, sw_ref, out_hbm,
             acc_ref, oxa, oxb, owa, owb,
             cxr_ref, cwr_ref, cxl_ref, cwl_ref,
             rxs, rxr, rws, rwr, lxs, lxr, lws, lwr,
             out_sems, credit_cw, credit_ccw):
        my = lax.axis_index("i")
        left = lax.rem(my + N_DEV - 1, N_DEV)
        right = lax.rem(my + 1, N_DEV)

        barrier_sem = pltpu.get_barrier_semaphore()
        for nbr in (left, right):
            pl.semaphore_signal(
                barrier_sem, inc=1,
                device_id=(nbr,), device_id_type=pl.DeviceIdType.MESH,
            )
        pl.semaphore_wait(barrier_sem, 2)

        oxa[...] = x_ref[:, :kh].astype(jnp.float8_e5m2)
        oxb[...] = x_ref[:, kh:].astype(jnp.float8_e5m2)
        owa[...] = w_ref[:kh, :].astype(jnp.float8_e5m2)
        owb[...] = w_ref[kh:, :].astype(jnp.float8_e5m2)

        def accum_gemm(xc, wc, first):
            def tile_body(i, _):
                sl = pl.ds(i * M_TILE, M_TILE)
                part = jnp.dot(xc[sl, :], wc[:, :],
                               preferred_element_type=jnp.float32)
                if first:
                    acc_ref[sl, :] = part
                else:
                    acc_ref[sl, :] = acc_ref[sl, :] + part
                return 0
            lax.fori_loop(0, n_tiles, tile_body, 0)

        def rdma(src, dst, ssem, rsem, target):
            return pltpu.make_async_remote_copy(
                src_ref=src, dst_ref=dst, send_sem=ssem, recv_sem=rsem,
                device_id=(target,), device_id_type=pl.DeviceIdType.MESH,
            )

        dirs = (
            dict(cx=cxr_ref, cw=cwr_ref, xs=rxs, xr=rxr, ws=rws, wr=rwr,
                 tgt=right, own_x=oxa, own_w=owa),
            dict(cx=cxl_ref, cw=cwl_ref, xs=lxs, xr=lxr, ws=lws, wr=lwr,
                 tgt=left, own_x=oxb, own_w=owb),
        )

        def make_hop_sends(h):
            out = []
            dst = h % N_SLOT
            for d in dirs:
                if h == 0:
                    src_w = d["own_w"]
                    src_x0 = d["own_x"].at[pl.ds(0, MH), :]
                    src_x1 = d["own_x"].at[pl.ds(MH, MH), :]
                else:
                    p = (h - 1) % N_SLOT
                    src_w = d["cw"].at[p]
                    src_x0 = d["cx"].at[p, pl.ds(0, MH), :]
                    src_x1 = d["cx"].at[p, pl.ds(MH, MH), :]
                cw_c = rdma(src_w, d["cw"].at[dst],
                            d["ws"].at[dst], d["wr"].at[dst], d["tgt"])
                x0_c = rdma(src_x0,
                            d["cx"].at[dst, pl.ds(0, MH), :],
                            d["xs"].at[2 * dst], d["xr"].at[2 * dst],
                            d["tgt"])
                x1_c = rdma(src_x1,
                            d["cx"].at[dst, pl.ds(MH, MH), :],
                            d["xs"].at[2 * dst + 1], d["xr"].at[2 * dst + 1],
                            d["tgt"])
                out.append((cw_c, x0_c, x1_c))
            return out

        inflight = {}
        hop0 = make_hop_sends(0)
        for trio in hop0:
            for c in trio:
                c.start()
        inflight[0] = hop0

        for h in range(N_HOP):
            if h == 0:
                accum_gemm(oxa, owa, first=True)
                accum_gemm(oxb, owb, first=False)
            if 2 <= h <= 5:
                pl.semaphore_wait(credit_cw, 1)
                pl.semaphore_wait(credit_ccw, 1)
            if h < N_HOP - 1:
                nxt = make_hop_sends(h + 1)
                for di in range(2):
                    for pc in range(3):
                        inflight[h][di][pc].wait_recv()
                        nxt[di][pc].start()
                inflight[h + 1] = nxt
            else:
                for di in range(2):
                    inflight[h][di][0].wait_recv()
                    inflight[h][di][1].wait_recv()
            if h >= 1:
                p = (h - 1) % N_SLOT
                accum_gemm(cxr_ref.at[p], cwr_ref.at[p], first=False)
                accum_gemm(cxl_ref.at[p], cwl_ref.at[p], first=False)
            for di in range(2):
                for pc in range(3):
                    inflight[h][di][pc].wait_send()
            if 1 <= h <= 4:
                pl.semaphore_signal(
                    credit_cw, inc=1,
                    device_id=(left,), device_id_type=pl.DeviceIdType.MESH,
                )
                pl.semaphore_signal(
                    credit_ccw, inc=1,
                    device_id=(right,), device_id_type=pl.DeviceIdType.MESH,
                )

        p = (N_HOP - 1) % N_SLOT
        s = sx_ref[0] * sw_ref[0]
        out_copies = []

        def tail_tile(i):
            sl = pl.ds(i * M_TILE, M_TILE)
            acc = (acc_ref[sl, :]
                   + jnp.dot(cxr_ref[p, sl, :], cwr_ref[p, :, :],
                             preferred_element_type=jnp.float32)
                   + jnp.dot(cxl_ref[p, sl, :], cwl_ref[p, :, :],
                             preferred_element_type=jnp.float32))
            y = acc * s
            acc_ref[sl, :] = y * jax.nn.sigmoid(y)
            if len(out_copies) >= 4:
                out_copies[-4].wait()
            cp = pltpu.make_async_copy(
                acc_ref.at[sl, :], out_hbm.at[sl, :], out_sems.at[i % 4])
            cp.start()
            out_copies.append(cp)

        for i in range(n_tiles // 2):
            tail_tile(i)
        for di in range(2):
            inflight[N_HOP - 1][di][2].wait_recv()
        for i in range(n_tiles // 2, n_tiles):
            tail_tile(i)
        for cp in out_copies[-4:]:
            cp.wait()

    return pl.pallas_call(
        body,
        out_shape=jax.ShapeDtypeStruct((m, n), jnp.float32),
        in_specs=[
            pl.BlockSpec(memory_space=pltpu.VMEM),
            pl.BlockSpec(memory_space=pltpu.VMEM),
            pl.BlockSpec(memory_space=pltpu.SMEM),
            pl.BlockSpec(memory_space=pltpu.SMEM),
        ],
        out_specs=pl.BlockSpec(memory_space=pl.ANY),
        scratch_shapes=[
            pltpu.VMEM((m, n), jnp.float32),
            pltpu.VMEM((m, kh), jnp.float8_e5m2),
            pltpu.VMEM((m, kh), jnp.float8_e5m2),
            pltpu.VMEM((kh, n), jnp.float8_e5m2),
            pltpu.VMEM((kh, n), jnp.float8_e5m2),
            pltpu.VMEM((N_SLOT, m, kh), jnp.float8_e5m2),
            pltpu.VMEM((N_SLOT, kh, n), jnp.float8_e5m2),
            pltpu.VMEM((N_SLOT, m, kh), jnp.float8_e5m2),
            pltpu.VMEM((N_SLOT, kh, n), jnp.float8_e5m2),
            pltpu.SemaphoreType.DMA((2 * N_SLOT,)),
            pltpu.SemaphoreType.DMA((2 * N_SLOT,)),
            pltpu.SemaphoreType.DMA((N_SLOT,)),
            pltpu.SemaphoreType.DMA((N_SLOT,)),
            pltpu.SemaphoreType.DMA((2 * N_SLOT,)),
            pltpu.SemaphoreType.DMA((2 * N_SLOT,)),
            pltpu.SemaphoreType.DMA((N_SLOT,)),
            pltpu.SemaphoreType.DMA((N_SLOT,)),
            pltpu.SemaphoreType.DMA((4,)),
            pltpu.SemaphoreType.REGULAR,
            pltpu.SemaphoreType.REGULAR,
        ],
        compiler_params=pltpu.CompilerParams(
            collective_id=0,
            vmem_limit_bytes=62 * 1024 * 1024,
        ),
    )(x, w_mat, scale_x, scale_w)
